# Initial kernel scaffold; baseline (speedup 1.0000x reference)
import jax
import jax.numpy as jnp
from jax.experimental import pallas as pl


def kernel(x, edge_index, W1, b1, g1, be1, W2, b2, g2, be2, W3, b3):
    raise NotImplementedError("write your pallas kernel here")


# speedup vs baseline: 9.9391x; 9.9391x over previous
import functools

import jax
import jax.numpy as jnp
from jax import lax
from jax.experimental import pallas as pl
from jax.experimental.pallas import tpu as pltpu
from jax.experimental.pallas import tpu_sc as plsc

N = 50000
E = 800000
IN_DIM = 22
HID = 64
OUT = 32

NP_ = 50048
RPT = NP_ // 16
CH = 128
NCHUNKS = E // CH

_f32 = jnp.float32


def _mesh():
    return plsc.VectorSubcoreMesh(
        core_axis_name="c", subcore_axis_name="s", num_cores=2, num_subcores=16)


@functools.partial(
    pl.kernel,
    out_type=[jax.ShapeDtypeStruct((NP_,), _f32),
              jax.ShapeDtypeStruct((NP_,), _f32)],
    mesh=_mesh(),
    compiler_params=pltpu.CompilerParams(use_tc_tiling_on_sc=False),
    scratch_types=[pltpu.VMEM_SHARED((NP_,), _f32),
                   pltpu.VMEM((CH,), jnp.int32),
                   pltpu.VMEM((CH,), _f32),
                   pltpu.VMEM((RPT,), _f32)],
)
def _deg_kernel(ei, zvec, ones_in, d0, d1, acc, didx, ones, buf):
    s = lax.axis_index("s")
    c = lax.axis_index("c")
    wid = s * 2 + c
    row0 = s * RPT
    pltpu.sync_copy(zvec, buf)
    pltpu.sync_copy(buf, acc.at[pl.ds(row0, RPT)])
    pltpu.sync_copy(ones_in, ones)
    plsc.subcore_barrier()

    nch = 195 + jnp.where(wid < 10, 1, 0)

    def chunk(i, carry):
        off = (wid + i * 32) * CH
        pltpu.sync_copy(ei.at[1, pl.ds(off, CH)], didx)
        pltpu.sync_copy(ones, acc.at[didx], add=True)
        return carry

    lax.fori_loop(0, nch, chunk, 0)
    plsc.subcore_barrier()

    pltpu.sync_copy(acc.at[pl.ds(row0, RPT)], buf)

    @pl.when(c == 0)
    def _():
        pltpu.sync_copy(buf, d0.at[pl.ds(row0, RPT)])

    @pl.when(c == 1)
    def _():
        pltpu.sync_copy(buf, d1.at[pl.ds(row0, RPT)])


def _make_agg(dsc):
    @functools.partial(
        pl.kernel,
        out_type=[jax.ShapeDtypeStruct((NP_, dsc), _f32),
                  jax.ShapeDtypeStruct((NP_, dsc), _f32)],
        mesh=_mesh(),
        compiler_params=pltpu.CompilerParams(use_tc_tiling_on_sc=False),
        scratch_types=[pltpu.VMEM_SHARED((NP_, dsc), _f32),
                       pltpu.VMEM((CH,), jnp.int32),
                       pltpu.VMEM((CH,), jnp.int32),
                       pltpu.VMEM((CH, dsc), _f32),
                       pltpu.VMEM((184, dsc), _f32),
                       pltpu.SemaphoreType.DMA],
    )
    def agg(ei, t_lo, t_hi, zrows, out_lo, out_hi,
            acc, sidx, didx, rows, buf, sem):
        s = lax.axis_index("s")
        c = lax.axis_index("c")
        row0 = s * RPT
        pltpu.sync_copy(zrows, buf)
        for k in range(17):
            pltpu.sync_copy(buf, acc.at[pl.ds(row0 + k * 184, 184)])
        plsc.subcore_barrier()

        nch = 390 + jnp.where(s < 10, 1, 0)

        def run(t_ref):
            def chunk(i, carry):
                off = (s + i * 16) * CH
                pltpu.sync_copy(ei.at[0, pl.ds(off, CH)], sidx)
                pltpu.sync_copy(ei.at[1, pl.ds(off, CH)], didx)
                pltpu.async_copy(t_ref.at[sidx], rows, sem).wait()
                pltpu.sync_copy(rows, acc.at[didx], add=True)
                return carry
            lax.fori_loop(0, nch, chunk, 0)

        @pl.when(c == 0)
        def _():
            run(t_lo)

        @pl.when(c == 1)
        def _():
            run(t_hi)

        plsc.subcore_barrier()

        def copy_out(out_ref):
            for k in range(17):
                sl = pl.ds(row0 + k * 184, 184)
                pltpu.sync_copy(acc.at[sl], buf)
                pltpu.sync_copy(buf, out_ref.at[sl])

        @pl.when(c == 0)
        def _():
            copy_out(out_lo)

        @pl.when(c == 1)
        def _():
            copy_out(out_hi)

    return agg


_agg16 = _make_agg(16)
_agg32 = _make_agg(32)


BR = 2000
GRID = N // BR


def _row_spec(cols):
    return pl.BlockSpec((BR, cols), lambda i: (i, 0))


def _full_spec(rows, cols):
    return pl.BlockSpec((rows, cols), lambda i: (0, 0))


def _prep_body(d0_ref, d1_ref, x_ref, dinv_ref, tlo_ref, thi_ref):
    deg = d0_ref[...] + d1_ref[...] + 1.0
    dinv = lax.rsqrt(deg)
    dinv_ref[...] = dinv
    t = x_ref[...] * dinv
    tlo_ref[...] = t[:, :16]
    thi_ref[...] = jnp.concatenate(
        [t[:, 16:IN_DIM], jnp.zeros((BR, 32 - IN_DIM), _f32)], axis=1)


_prep = pl.pallas_call(
    _prep_body,
    grid=(GRID,),
    in_specs=[_row_spec(1), _row_spec(1), _row_spec(IN_DIM)],
    out_specs=[_row_spec(1), _row_spec(16), _row_spec(16)],
    out_shape=[jax.ShapeDtypeStruct((N, 1), _f32),
               jax.ShapeDtypeStruct((N, 16), _f32),
               jax.ShapeDtypeStruct((N, 16), _f32)],
)


def _make_lin(dsc, din_pad, dout):
    def body(alo_ref, ahi_ref, tlo_ref, thi_ref, dinv_ref, w_ref, b_ref,
             y_ref, st_ref):
        i = pl.program_id(0)
        dinv = dinv_ref[...]
        u = jnp.concatenate(
            [dinv * (alo_ref[...] + tlo_ref[...]),
             dinv * (ahi_ref[...] + thi_ref[...])], axis=1)
        y = jnp.dot(u, w_ref[...], preferred_element_type=_f32) + b_ref[...]
        y_ref[...] = y

        @pl.when(i == 0)
        def _():
            st_ref[...] = jnp.zeros((8, dout), _f32)

        st_ref[...] += jnp.concatenate(
            [jnp.sum(y, axis=0, keepdims=True),
             jnp.sum(y * y, axis=0, keepdims=True),
             jnp.zeros((6, dout), _f32)], axis=0)

    return pl.pallas_call(
        body,
        grid=(GRID,),
        in_specs=[_row_spec(dsc), _row_spec(dsc), _row_spec(dsc),
                  _row_spec(dsc), _row_spec(1),
                  _full_spec(din_pad, dout), _full_spec(1, dout)],
        out_specs=[_row_spec(dout), _full_spec(8, dout)],
        out_shape=[jax.ShapeDtypeStruct((N, dout), _f32),
                   jax.ShapeDtypeStruct((8, dout), _f32)],
    )


_lin1 = _make_lin(16, 32, HID)
_lin2 = _make_lin(32, HID, HID)


def _make_bn(dout, with_w3):
    def body(*refs):
        if with_w3:
            (y_ref, st_ref, g_ref, be_ref, dinv_ref, w3_ref,
             tlo_ref, thi_ref) = refs
        else:
            (y_ref, st_ref, g_ref, be_ref, dinv_ref,
             tlo_ref, thi_ref) = refs
        mu = st_ref[0:1, :] / N
        var = st_ref[1:2, :] / N - mu * mu
        scale = g_ref[...] * lax.rsqrt(var + 1e-5)
        h = jnp.maximum((y_ref[...] - mu) * scale + be_ref[...], 0.0)
        if with_w3:
            h = jnp.dot(h, w3_ref[...], preferred_element_type=_f32)
        t = h * dinv_ref[...]
        half = t.shape[1] // 2
        tlo_ref[...] = t[:, :half]
        thi_ref[...] = t[:, half:]

    dnext = OUT if with_w3 else dout
    in_specs = [_row_spec(dout), _full_spec(8, dout), _full_spec(1, dout),
                _full_spec(1, dout), _row_spec(1)]
    if with_w3:
        in_specs.append(_full_spec(HID, OUT))
    return pl.pallas_call(
        body,
        grid=(GRID,),
        in_specs=in_specs,
        out_specs=[_row_spec(dnext // 2), _row_spec(dnext // 2)],
        out_shape=[jax.ShapeDtypeStruct((N, dnext // 2), _f32),
                   jax.ShapeDtypeStruct((N, dnext // 2), _f32)],
    )


_bn1 = _make_bn(HID, False)
_bn2 = _make_bn(HID, True)


def _final_body(alo_ref, ahi_ref, tlo_ref, thi_ref, dinv_ref, b_ref, o_ref):
    dinv = dinv_ref[...]
    u = jnp.concatenate(
        [dinv * (alo_ref[...] + tlo_ref[...]),
         dinv * (ahi_ref[...] + thi_ref[...])], axis=1)
    o_ref[...] = jnp.maximum(u + b_ref[...], 0.0)


_final = pl.pallas_call(
    _final_body,
    grid=(GRID,),
    in_specs=[_row_spec(16), _row_spec(16), _row_spec(16), _row_spec(16),
              _row_spec(1), _full_spec(1, OUT)],
    out_specs=_row_spec(OUT),
    out_shape=jax.ShapeDtypeStruct((N, OUT), _f32),
)


def kernel(x, edge_index, W1, b1, g1, be1, W2, b2, g2, be2, W3, b3):
    ei = edge_index.astype(jnp.int32)

    zvec = jnp.zeros((RPT,), _f32)
    ones_in = jnp.ones((CH,), _f32)
    z16 = jnp.zeros((184, 16), _f32)
    z32 = jnp.zeros((184, 32), _f32)

    d0, d1 = _deg_kernel(ei, zvec, ones_in)
    dinv, t1l, t1h = _prep(d0.reshape(NP_, 1)[:N], d1.reshape(NP_, 1)[:N], x)

    a1l, a1h = _agg16(ei, t1l, t1h, z16)
    W1p = jnp.concatenate([W1, jnp.zeros((32 - IN_DIM, HID), _f32)], axis=0)
    y1, st1 = _lin1(a1l[:N], a1h[:N], t1l, t1h, dinv, W1p, b1.reshape(1, -1))
    t2l, t2h = _bn1(y1, st1, g1.reshape(1, -1), be1.reshape(1, -1), dinv)

    a2l, a2h = _agg32(ei, t2l, t2h, z32)
    y2, st2 = _lin2(a2l[:N], a2h[:N], t2l, t2h, dinv, W2, b2.reshape(1, -1))
    t3l, t3h = _bn2(y2, st2, g2.reshape(1, -1), be2.reshape(1, -1), dinv, W3)

    a3l, a3h = _agg16(ei, t3l, t3h, z16)
    out = _final(a3l[:N], a3h[:N], t3l, t3h, dinv, b3.reshape(1, -1))
    return out


# speedup vs baseline: 17.7432x; 1.7852x over previous
import functools

import jax
import jax.numpy as jnp
from jax import lax
from jax.experimental import pallas as pl
from jax.experimental.pallas import tpu as pltpu
from jax.experimental.pallas import tpu_sc as plsc

N = 50000
E = 800000
IN_DIM = 22
HID = 64
OUT = 32

NP_ = 50048
RPT = NP_ // 16
CH = 128
BLK = 5
CPT = 400
E_PAD = CPT * 16 * CH
NBLK = CPT // BLK
DCPW = E_PAD // (32 * CH)
DNBLK = DCPW // BLK

_f32 = jnp.float32


def _mesh():
    return plsc.VectorSubcoreMesh(
        core_axis_name="c", subcore_axis_name="s", num_cores=2, num_subcores=16)


@functools.partial(
    pl.kernel,
    out_type=[jax.ShapeDtypeStruct((NP_,), _f32),
              jax.ShapeDtypeStruct((NP_,), _f32)],
    mesh=_mesh(),
    compiler_params=pltpu.CompilerParams(use_tc_tiling_on_sc=False),
    scratch_types=[pltpu.VMEM_SHARED((NP_,), _f32),
                   pltpu.VMEM((4, BLK * CH), jnp.int32),
                   pltpu.VMEM((BLK * CH,), _f32),
                   pltpu.VMEM((RPT,), _f32),
                   pltpu.SemaphoreType.DMA((2,)),
                   pltpu.SemaphoreType.DMA((4,))],
)
def _deg_kernel(dst2, zvec, ones_in, d0, d1, acc, ibd, ones, buf, ssem, isem):
    s = lax.axis_index("s")
    c = lax.axis_index("c")
    wid = s * 2 + c
    row0 = s * RPT
    pltpu.sync_copy(zvec, buf)
    pltpu.sync_copy(buf, acc.at[pl.ds(row0, RPT)])
    pltpu.sync_copy(ones_in, ones)
    plsc.subcore_barrier()

    C0B = wid * DNBLK

    def i_issue(blk, hh):
        pltpu.async_copy(dst2.at[C0B + blk], ibd.at[hh], isem.at[hh])

    def i_wait(hh):
        pltpu.make_async_copy(dst2.at[C0B], ibd.at[hh], isem.at[hh]).wait()

    def s_issue(hh, q):
        pltpu.async_copy(ones, acc.at[ibd.at[hh]], ssem.at[q], add=True)

    def s_wait(hh, q):
        pltpu.make_async_copy(ones, acc.at[ibd.at[hh]], ssem.at[q]).wait()

    pltpu.sync_copy(dst2.at[C0B], ibd.at[0])
    pltpu.sync_copy(dst2.at[C0B + 1], ibd.at[1])

    def body(mo, carry):
        for j in range(4):
            m = mo * 4 + j
            q = j % 2
            s_issue(j, q)

            @pl.when(m >= 1)
            def _():
                s_wait((j + 3) % 4, 1 - q)

            @pl.when(m + 2 < DNBLK)
            def _():
                i_issue(m + 2, (j + 2) % 4)

            @pl.when(jnp.logical_and(m >= 1, m + 1 < DNBLK))
            def _():
                i_wait((j + 1) % 4)
        return carry

    lax.fori_loop(0, DNBLK // 4, body, 0)
    s_wait(3, 1)
    plsc.subcore_barrier()

    pltpu.sync_copy(acc.at[pl.ds(row0, RPT)], buf)

    @pl.when(c == 0)
    def _():
        pltpu.sync_copy(buf, d0.at[pl.ds(row0, RPT)])

    @pl.when(c == 1)
    def _():
        pltpu.sync_copy(buf, d1.at[pl.ds(row0, RPT)])


def _make_agg(dsc, blk):
    @functools.partial(
        pl.kernel,
        out_type=[jax.ShapeDtypeStruct((NP_, dsc), _f32),
                  jax.ShapeDtypeStruct((NP_, dsc), _f32)],
        mesh=_mesh(),
        compiler_params=pltpu.CompilerParams(use_tc_tiling_on_sc=False),
        scratch_types=[pltpu.VMEM_SHARED((NP_, dsc), _f32),
                       pltpu.VMEM((4, blk * CH), jnp.int32),
                       pltpu.VMEM((4, blk * CH), jnp.int32),
                       pltpu.VMEM((2, blk * CH, dsc), _f32),
                       pltpu.VMEM((184, dsc), _f32),
                       pltpu.SemaphoreType.DMA((2,)),
                       pltpu.SemaphoreType.DMA((2,)),
                       pltpu.SemaphoreType.DMA((4,)),
                       pltpu.SemaphoreType.DMA((4,))],
    )
    def agg(src2, dst2, t_lo, t_hi, zrows, out_lo, out_hi,
            acc, ibs, ibd, rows, buf, gsem, ssem, isrc, isdt):
        s = lax.axis_index("s")
        c = lax.axis_index("c")
        row0 = s * RPT
        pltpu.sync_copy(zrows, buf)
        for k in range(17):
            pltpu.sync_copy(buf, acc.at[pl.ds(row0 + k * 184, 184)])
        plsc.subcore_barrier()

        nblk = CPT // blk
        C0B = s * nblk

        def run(t_ref):
            def g_issue(hh, q):
                pltpu.async_copy(
                    t_ref.at[ibs.at[hh]], rows.at[q], gsem.at[q])

            def g_wait(hh, q):
                pltpu.make_async_copy(
                    t_ref.at[ibs.at[hh]], rows.at[q], gsem.at[q]).wait()

            def s_issue(hh, q):
                pltpu.async_copy(
                    rows.at[q], acc.at[ibd.at[hh]], ssem.at[q], add=True)

            def s_wait(hh, q):
                pltpu.make_async_copy(
                    rows.at[q], acc.at[ibd.at[hh]], ssem.at[q]).wait()

            def i_issue(blk, hh):
                pltpu.async_copy(src2.at[C0B + blk], ibs.at[hh], isrc.at[hh])
                pltpu.async_copy(dst2.at[C0B + blk], ibd.at[hh], isdt.at[hh])

            def i_wait(hh):
                pltpu.make_async_copy(src2.at[C0B], ibs.at[hh],
                                      isrc.at[hh]).wait()
                pltpu.make_async_copy(dst2.at[C0B], ibd.at[hh],
                                      isdt.at[hh]).wait()

            pltpu.sync_copy(src2.at[C0B], ibs.at[0])
            pltpu.sync_copy(dst2.at[C0B], ibd.at[0])
            pltpu.sync_copy(src2.at[C0B + 1], ibs.at[1])
            pltpu.sync_copy(dst2.at[C0B + 1], ibd.at[1])
            g_issue(0, 0)

            def body(mo, carry):
                for j in range(4):
                    m = mo * 4 + j
                    q = j % 2
                    g_wait(j, q)
                    s_issue(j, q)

                    @pl.when(m >= 1)
                    def _():
                        s_wait((j + 3) % 4, 1 - q)

                    @pl.when(m + 2 < nblk)
                    def _():
                        i_issue(m + 2, (j + 2) % 4)

                    @pl.when(jnp.logical_and(m >= 1, m + 1 < nblk))
                    def _():
                        i_wait((j + 1) % 4)

                    @pl.when(m + 1 < nblk)
                    def _():
                        g_issue((j + 1) % 4, 1 - q)
                return carry

            lax.fori_loop(0, nblk // 4, body, 0)
            s_wait(3, 1)

        @pl.when(c == 0)
        def _():
            run(t_lo)

        @pl.when(c == 1)
        def _():
            run(t_hi)

        plsc.subcore_barrier()

        def copy_out(out_ref):
            for k in range(17):
                sl = pl.ds(row0 + k * 184, 184)
                pltpu.sync_copy(acc.at[sl], buf)
                pltpu.sync_copy(buf, out_ref.at[sl])

        @pl.when(c == 0)
        def _():
            copy_out(out_lo)

        @pl.when(c == 1)
        def _():
            copy_out(out_hi)

    return agg


_agg16 = _make_agg(16, 5)
_agg32 = _make_agg(32, 2)


BR = 2000
GRID = N // BR


def _row_spec(cols):
    return pl.BlockSpec((BR, cols), lambda i: (i, 0))


def _full_spec(rows, cols):
    return pl.BlockSpec((rows, cols), lambda i: (0, 0))


def _prep_body(d0_ref, d1_ref, x_ref, dinv_ref, tlo_ref, thi_ref):
    deg = d0_ref[...] + d1_ref[...] + 1.0
    dinv = lax.rsqrt(deg)
    dinv_ref[...] = dinv
    t = x_ref[...] * dinv
    tlo_ref[...] = t[:, :16]
    thi_ref[...] = jnp.concatenate(
        [t[:, 16:IN_DIM], jnp.zeros((BR, 32 - IN_DIM), _f32)], axis=1)


_prep = pl.pallas_call(
    _prep_body,
    grid=(GRID,),
    in_specs=[_row_spec(1), _row_spec(1), _row_spec(IN_DIM)],
    out_specs=[_row_spec(1), _row_spec(16), _row_spec(16)],
    out_shape=[jax.ShapeDtypeStruct((N, 1), _f32),
               jax.ShapeDtypeStruct((N, 16), _f32),
               jax.ShapeDtypeStruct((N, 16), _f32)],
)


def _make_lin(dsc, din_pad, dout):
    def body(alo_ref, ahi_ref, tlo_ref, thi_ref, dinv_ref, w_ref, b_ref,
             y_ref, st_ref):
        i = pl.program_id(0)
        dinv = dinv_ref[...]
        u = jnp.concatenate(
            [dinv * (alo_ref[...] + tlo_ref[...]),
             dinv * (ahi_ref[...] + thi_ref[...])], axis=1)
        y = jnp.dot(u, w_ref[...], preferred_element_type=_f32) + b_ref[...]
        y_ref[...] = y

        @pl.when(i == 0)
        def _():
            st_ref[...] = jnp.zeros((8, dout), _f32)

        st_ref[...] += jnp.concatenate(
            [jnp.sum(y, axis=0, keepdims=True),
             jnp.sum(y * y, axis=0, keepdims=True),
             jnp.zeros((6, dout), _f32)], axis=0)

    return pl.pallas_call(
        body,
        grid=(GRID,),
        in_specs=[_row_spec(dsc), _row_spec(dsc), _row_spec(dsc),
                  _row_spec(dsc), _row_spec(1),
                  _full_spec(din_pad, dout), _full_spec(1, dout)],
        out_specs=[_row_spec(dout), _full_spec(8, dout)],
        out_shape=[jax.ShapeDtypeStruct((N, dout), _f32),
                   jax.ShapeDtypeStruct((8, dout), _f32)],
    )


_lin1 = _make_lin(16, 32, HID)
_lin2 = _make_lin(32, HID, HID)


def _make_bn(dout, with_w3):
    def body(*refs):
        if with_w3:
            (y_ref, st_ref, g_ref, be_ref, dinv_ref, w3_ref,
             tlo_ref, thi_ref) = refs
        else:
            (y_ref, st_ref, g_ref, be_ref, dinv_ref,
             tlo_ref, thi_ref) = refs
        mu = st_ref[0:1, :] / N
        var = st_ref[1:2, :] / N - mu * mu
        scale = g_ref[...] * lax.rsqrt(var + 1e-5)
        h = jnp.maximum((y_ref[...] - mu) * scale + be_ref[...], 0.0)
        if with_w3:
            h = jnp.dot(h, w3_ref[...], preferred_element_type=_f32)
        t = h * dinv_ref[...]
        half = t.shape[1] // 2
        tlo_ref[...] = t[:, :half]
        thi_ref[...] = t[:, half:]

    dnext = OUT if with_w3 else dout
    in_specs = [_row_spec(dout), _full_spec(8, dout), _full_spec(1, dout),
                _full_spec(1, dout), _row_spec(1)]
    if with_w3:
        in_specs.append(_full_spec(HID, OUT))
    return pl.pallas_call(
        body,
        grid=(GRID,),
        in_specs=in_specs,
        out_specs=[_row_spec(dnext // 2), _row_spec(dnext // 2)],
        out_shape=[jax.ShapeDtypeStruct((N, dnext // 2), _f32),
                   jax.ShapeDtypeStruct((N, dnext // 2), _f32)],
    )


_bn1 = _make_bn(HID, False)
_bn2 = _make_bn(HID, True)


def _final_body(alo_ref, ahi_ref, tlo_ref, thi_ref, dinv_ref, b_ref, o_ref):
    dinv = dinv_ref[...]
    u = jnp.concatenate(
        [dinv * (alo_ref[...] + tlo_ref[...]),
         dinv * (ahi_ref[...] + thi_ref[...])], axis=1)
    o_ref[...] = jnp.maximum(u + b_ref[...], 0.0)


_final = pl.pallas_call(
    _final_body,
    grid=(GRID,),
    in_specs=[_row_spec(16), _row_spec(16), _row_spec(16), _row_spec(16),
              _row_spec(1), _full_spec(1, OUT)],
    out_specs=_row_spec(OUT),
    out_shape=jax.ShapeDtypeStruct((N, OUT), _f32),
)


def kernel(x, edge_index, W1, b1, g1, be1, W2, b2, g2, be2, W3, b3):
    ei = edge_index.astype(jnp.int32)

    npad = E_PAD - E
    src_pad = jnp.concatenate([ei[0], jnp.zeros((npad,), jnp.int32)])
    dst_pad = jnp.concatenate(
        [ei[1], N + (jnp.arange(npad, dtype=jnp.int32) % (NP_ - N))])
    src2 = src_pad.reshape(E_PAD // (5 * CH), 5 * CH)
    dst2 = dst_pad.reshape(E_PAD // (5 * CH), 5 * CH)
    src2b = src_pad.reshape(E_PAD // (2 * CH), 2 * CH)
    dst2b = dst_pad.reshape(E_PAD // (2 * CH), 2 * CH)

    zvec = jnp.zeros((RPT,), _f32)
    ones_in = jnp.ones((BLK * CH,), _f32)
    z16 = jnp.zeros((184, 16), _f32)
    z32 = jnp.zeros((184, 32), _f32)

    d0, d1 = _deg_kernel(dst2, zvec, ones_in)
    dinv, t1l, t1h = _prep(d0.reshape(NP_, 1)[:N], d1.reshape(NP_, 1)[:N], x)

    a1l, a1h = _agg16(src2, dst2, t1l, t1h, z16)
    W1p = jnp.concatenate([W1, jnp.zeros((32 - IN_DIM, HID), _f32)], axis=0)
    y1, st1 = _lin1(a1l[:N], a1h[:N], t1l, t1h, dinv, W1p, b1.reshape(1, -1))
    t2l, t2h = _bn1(y1, st1, g1.reshape(1, -1), be1.reshape(1, -1), dinv)

    a2l, a2h = _agg32(src2b, dst2b, t2l, t2h, z32)
    y2, st2 = _lin2(a2l[:N], a2h[:N], t2l, t2h, dinv, W2, b2.reshape(1, -1))
    t3l, t3h = _bn2(y2, st2, g2.reshape(1, -1), be2.reshape(1, -1), dinv, W3)

    a3l, a3h = _agg16(src2, dst2, t3l, t3h, z16)
    out = _final(a3l[:N], a3h[:N], t3l, t3h, dinv, b3.reshape(1, -1))
    return out
